# Initial kernel scaffold; baseline (speedup 1.0000x reference)
import jax
import jax.numpy as jnp
from jax.experimental import pallas as pl


def kernel(inputs, W_emb, init_w, in_proj_w, in_proj_b, out_proj_w, out_proj_b, ln1_g, ln1_b, ff_w1, ff_b1, ff_w2, ff_b2, ln2_g, ln2_b, hc_w, hc_b, vw_w, vw_b, pq_w, pq_b, pr_w, pr_b, p_v):
    raise NotImplementedError("write your pallas kernel here")


# speedup vs baseline: 1.7683x; 1.7683x over previous
import jax
import jax.numpy as jnp
from jax.experimental import pallas as pl
from jax.experimental.pallas import tpu as pltpu

_B, _S, _D, _H = 128, 100, 128, 8
_FF = 2048
_DH = _D // _H
_CS = 10.0

_BB_ENC = 16
_BB_DEC = 64


def _lane_sum(x):
    n = x.shape[-1]
    nfull = (n // 8) * 8
    acc = x[..., 0:8]
    for o in range(8, nfull, 8):
        acc = acc + x[..., o:o + 8]
    rem = n - nfull
    if rem:
        head = acc[..., :rem] + x[..., nfull:]
        acc = jnp.concatenate([head, acc[..., rem:]], axis=-1)
    t4 = acc[..., 0:4] + acc[..., 4:8]
    t2 = t4[..., 0:2] + t4[..., 2:4]
    return t2[..., 0:1] + t2[..., 1:2]


def _lnorm(x, g, b):
    m = _lane_sum(x) / jnp.float32(x.shape[-1])
    d = x - m
    v = _lane_sum(d * d) / jnp.float32(x.shape[-1])
    return d / jnp.sqrt(v + 1e-5) * g + b


def _encoder_body(inp_ref, ipw_ref, ipb_ref, opw_ref, opb_ref,
                  ln1g_ref, ln1b_ref, f1w_ref, f1b_ref, f2w_ref, f2b_ref,
                  ln2g_ref, ln2b_ref, hcw_ref, hcb_ref, initw_ref, vww_ref,
                  vwb_ref, prw_ref, prb_ref, wemb_ref,
                  h_ref, rp_ref, hbar_ref, q0_ref):
    bb = _BB_ENC
    e2 = jnp.dot(inp_ref[...], wemb_ref[...])
    qkv = jnp.dot(e2, ipw_ref[...]) + ipb_ref[...]
    q = qkv[:, :_D]
    k = qkv[:, _D:2 * _D]
    v = qkv[:, 2 * _D:]
    outs = []
    for hh in range(_H):
        sl = slice(hh * _DH, (hh + 1) * _DH)
        qh = q[:, sl].reshape(bb, _S, _DH)
        kh = k[:, sl].reshape(bb, _S, _DH)
        vh = v[:, sl].reshape(bb, _S, _DH)
        s = jax.lax.dot_general(qh, kh, (((2,), (2,)), ((0,), (0,))))
        s = s / jnp.sqrt(jnp.float32(_DH))
        m = jnp.max(s, axis=-1, keepdims=True)
        e = jnp.exp(s - m)
        att = e / _lane_sum(e)
        outs.append(jax.lax.dot_general(att, vh, (((2,), (1,)), ((0,), (0,)))))
    attn = jnp.concatenate(outs, axis=-1).reshape(bb * _S, _D)
    attn = jnp.dot(attn, opw_ref[...]) + opb_ref[...]
    x1 = _lnorm(e2 + attn, ln1g_ref[...], ln1b_ref[...])
    ffa = jnp.maximum(jnp.dot(x1, f1w_ref[...]) + f1b_ref[...], 0.0)
    ff = jnp.dot(ffa, f2w_ref[...]) + f2b_ref[...]
    h = _lnorm(x1 + ff, ln2g_ref[...], ln2b_ref[...])
    h3 = h.reshape(bb, _S, _D)
    h_ref[...] = h3
    rp_ref[...] = (jnp.dot(h, prw_ref[...]) + prb_ref[...]).reshape(bb, _S, _D)
    acc = h3[:, 0, :]
    for i in range(1, _S):
        acc = acc + h3[:, i, :]
    h_mean = acc / jnp.float32(_S)
    h_bar = jnp.dot(h_mean, hcw_ref[...]) + hcb_ref[...]
    hbar_ref[...] = h_bar
    q0_ref[...] = h_bar + (jnp.dot(initw_ref[...], vww_ref[...]) + vwb_ref[...])


def _decoder_body(h_ref, rp_ref, hbar_ref, q0_ref, g_ref, pqw_ref, pqb_ref,
                  pv_ref, vww_ref, vwb_ref, lps_ref, chs_ref):
    bb = _BB_DEC
    h = h_ref[...]
    rp = rp_ref[...]
    hbar = hbar_ref[...]
    pqw = pqw_ref[...]
    pqb = pqb_ref[...]
    pv = pv_ref[...]
    vww = vww_ref[...]
    vwb = vwb_ref[...]
    iota_s = jax.lax.broadcasted_iota(jnp.int32, (bb, _S), 1)

    def body(i, carry):
        query, maskf, first, lps_acc, chs_acc = carry
        qp = jnp.dot(query, pqw) + pqb
        t = jnp.tanh(qp[:, None, :] + rp)
        u1 = jnp.dot(t.reshape(bb * _S, _D), pv)
        u = jnp.sum(u1.reshape(bb, _S, 1), axis=-1)
        logits = _CS * jnp.tanh(u)
        logits = jnp.where(maskf > 0.5, -100000.0, logits)
        z = logits + g_ref[i]
        zm = jnp.max(z, axis=-1, keepdims=True)
        cand = jnp.where(z == zm, iota_s, _S)
        chosen = jnp.min(cand, axis=-1, keepdims=True)
        lmax = jnp.max(logits, axis=-1, keepdims=True)
        sh = logits - lmax
        lsm = sh - jnp.log(_lane_sum(jnp.exp(sh)))
        onehot = iota_s == chosen
        oh_f = onehot.astype(jnp.float32)
        logprob = jnp.sum(jnp.where(onehot, lsm, 0.0), axis=-1, keepdims=True)
        maskf = jnp.maximum(maskf, oh_f)
        chosen_hs = jax.lax.dot_general(
            oh_f, h, (((1,), (1,)), ((0,), (0,))))
        first = jnp.where(i == 0, chosen_hs, first)
        cat = jnp.concatenate([first, chosen_hs], axis=-1)
        query = hbar + (jnp.dot(cat, vww) + vwb)
        lps_acc = jnp.where(iota_s == i, logprob, lps_acc)
        chs_acc = jnp.where(iota_s == i, chosen, chs_acc)
        return query, maskf, first, lps_acc, chs_acc

    init = (q0_ref[...],
            jnp.zeros((bb, _S), jnp.float32),
            jnp.zeros((bb, _D), jnp.float32),
            jnp.zeros((bb, _S), jnp.float32),
            jnp.zeros((bb, _S), jnp.int32))
    _, _, _, lps_acc, chs_acc = jax.lax.fori_loop(0, _S, body, init)
    lps_ref[...] = lps_acc
    chs_ref[...] = chs_acc


def _full(shape):
    return pl.BlockSpec(shape, lambda i: (0,) * len(shape))


def kernel(inputs, W_emb, init_w, in_proj_w, in_proj_b, out_proj_w, out_proj_b,
           ln1_g, ln1_b, ff_w1, ff_b1, ff_w2, ff_b2, ln2_g, ln2_b,
           hc_w, hc_b, vw_w, vw_b, pq_w, pq_b, pr_w, pr_b, p_v):
    inp2 = inputs.reshape(_B * _S, 2)
    base_key = jax.random.key(42)
    g_tab = jnp.stack([
        jax.random.gumbel(jax.random.fold_in(base_key, i), (_B, _S), jnp.float32)
        for i in range(_S)
    ])

    r1 = lambda a: a.reshape(1, -1)
    nb = _B // _BB_ENC
    enc_specs = [
        pl.BlockSpec((_BB_ENC * _S, 2), lambda i: (i, 0)),
        _full((_D, 3 * _D)), _full((1, 3 * _D)),
        _full((_D, _D)), _full((1, _D)),
        _full((1, _D)), _full((1, _D)),
        _full((_D, _FF)), _full((1, _FF)),
        _full((_FF, _D)), _full((1, _D)),
        _full((1, _D)), _full((1, _D)),
        _full((_D, _D)), _full((1, _D)),
        _full((1, 2 * _D)),
        _full((2 * _D, _D)), _full((1, _D)),
        _full((_D, _D)), _full((1, _D)),
        _full((2, _D)),
    ]
    enc_out_specs = [
        pl.BlockSpec((_BB_ENC, _S, _D), lambda i: (i, 0, 0)),
        pl.BlockSpec((_BB_ENC, _S, _D), lambda i: (i, 0, 0)),
        pl.BlockSpec((_BB_ENC, _D), lambda i: (i, 0)),
        pl.BlockSpec((_BB_ENC, _D), lambda i: (i, 0)),
    ]
    h, rp, hbar, q0 = pl.pallas_call(
        _encoder_body,
        grid=(nb,),
        in_specs=enc_specs,
        out_specs=enc_out_specs,
        out_shape=[
            jax.ShapeDtypeStruct((_B, _S, _D), jnp.float32),
            jax.ShapeDtypeStruct((_B, _S, _D), jnp.float32),
            jax.ShapeDtypeStruct((_B, _D), jnp.float32),
            jax.ShapeDtypeStruct((_B, _D), jnp.float32),
        ],
        compiler_params=pltpu.CompilerParams(
            dimension_semantics=("parallel",)),
    )(inp2, in_proj_w.T, r1(in_proj_b), out_proj_w.T, r1(out_proj_b),
      r1(ln1_g), r1(ln1_b), ff_w1.T, r1(ff_b1), ff_w2.T, r1(ff_b2),
      r1(ln2_g), r1(ln2_b), hc_w.T, r1(hc_b), r1(init_w), vw_w.T, r1(vw_b),
      pr_w.T, r1(pr_b), W_emb)

    nd = _B // _BB_DEC
    dec_specs = [
        pl.BlockSpec((_BB_DEC, _S, _D), lambda j: (j, 0, 0)),
        pl.BlockSpec((_BB_DEC, _S, _D), lambda j: (j, 0, 0)),
        pl.BlockSpec((_BB_DEC, _D), lambda j: (j, 0)),
        pl.BlockSpec((_BB_DEC, _D), lambda j: (j, 0)),
        pl.BlockSpec((_S, _BB_DEC, _S), lambda j: (0, j, 0)),
        _full((_D, _D)), _full((1, _D)),
        _full((_D, 1)),
        _full((2 * _D, _D)), _full((1, _D)),
    ]
    dec_out_specs = [
        pl.BlockSpec((_BB_DEC, _S), lambda j: (j, 0)),
        pl.BlockSpec((_BB_DEC, _S), lambda j: (j, 0)),
    ]
    lps, chs = pl.pallas_call(
        _decoder_body,
        grid=(nd,),
        in_specs=dec_specs,
        out_specs=dec_out_specs,
        out_shape=[
            jax.ShapeDtypeStruct((_B, _S), jnp.float32),
            jax.ShapeDtypeStruct((_B, _S), jnp.int32),
        ],
        compiler_params=pltpu.CompilerParams(
            dimension_semantics=("parallel",)),
    )(h, rp, hbar, q0, g_tab, pq_w.T, r1(pq_b), p_v.reshape(_D, 1), vw_w.T,
      r1(vw_b))
    return lps, chs


# speedup vs baseline: 1.7694x; 1.0006x over previous
import jax
import jax.numpy as jnp
from jax.experimental import pallas as pl
from jax.experimental.pallas import tpu as pltpu

_B, _S, _D, _H = 128, 100, 128, 8
_FF = 2048
_DH = _D // _H
_CS = 10.0

_BB_ENC = 16
_BB_DEC = 64


def _lane_sum(x):
    n = x.shape[-1]
    nfull = (n // 8) * 8
    acc = x[..., 0:8]
    for o in range(8, nfull, 8):
        acc = acc + x[..., o:o + 8]
    rem = n - nfull
    if rem:
        head = acc[..., :rem] + x[..., nfull:]
        acc = jnp.concatenate([head, acc[..., rem:]], axis=-1)
    t4 = acc[..., 0:4] + acc[..., 4:8]
    t2 = t4[..., 0:2] + t4[..., 2:4]
    return t2[..., 0:1] + t2[..., 1:2]


def _lnorm(x, g, b):
    m = _lane_sum(x) / jnp.float32(x.shape[-1])
    d = x - m
    v = _lane_sum(d * d) / jnp.float32(x.shape[-1])
    return d / jnp.sqrt(v + 1e-5) * g + b


def _encoder_body(inp_ref, ipw_ref, ipb_ref, opw_ref, opb_ref,
                  ln1g_ref, ln1b_ref, f1w_ref, f1b_ref, f2w_ref, f2b_ref,
                  ln2g_ref, ln2b_ref, hcw_ref, hcb_ref, initw_ref, vww_ref,
                  vwb_ref, prw_ref, prb_ref, wemb_ref,
                  h_ref, rp_ref, hbar_ref, q0_ref):
    bb = _BB_ENC
    e2 = jnp.dot(inp_ref[...], wemb_ref[...])
    qkv = jnp.dot(e2, ipw_ref[...]) + ipb_ref[...]
    q = qkv[:, :_D]
    k = qkv[:, _D:2 * _D]
    v = qkv[:, 2 * _D:]
    outs = []
    for hh in range(_H):
        sl = slice(hh * _DH, (hh + 1) * _DH)
        qh = q[:, sl].reshape(bb, _S, _DH)
        kh = k[:, sl].reshape(bb, _S, _DH)
        vh = v[:, sl].reshape(bb, _S, _DH)
        s = jax.lax.dot_general(qh, kh, (((2,), (2,)), ((0,), (0,))))
        s = s / jnp.sqrt(jnp.float32(_DH))
        m = jnp.max(s, axis=-1, keepdims=True)
        e = jnp.exp(s - m)
        att = e / _lane_sum(e)
        outs.append(jax.lax.dot_general(att, vh, (((2,), (1,)), ((0,), (0,)))))
    attn = jnp.concatenate(outs, axis=-1).reshape(bb * _S, _D)
    attn = jnp.dot(attn, opw_ref[...]) + opb_ref[...]
    x1 = _lnorm(e2 + attn, ln1g_ref[...], ln1b_ref[...])
    ffa = jnp.maximum(jnp.dot(x1, f1w_ref[...]) + f1b_ref[...], 0.0)
    f2w = f2w_ref[...]
    ff = jnp.dot(ffa[:, 0:256], f2w[0:256, :])
    for o in range(256, _FF, 256):
        ff = ff + jnp.dot(ffa[:, o:o + 256], f2w[o:o + 256, :])
    ff = ff + f2b_ref[...]
    h = _lnorm(x1 + ff, ln2g_ref[...], ln2b_ref[...])
    h3 = h.reshape(bb, _S, _D)
    h_ref[...] = h3
    rp_ref[...] = (jnp.dot(h, prw_ref[...]) + prb_ref[...]).reshape(bb, _S, _D)
    acc = h3[:, 0, :]
    for i in range(1, _S):
        acc = acc + h3[:, i, :]
    h_mean = acc / jnp.float32(_S)
    h_bar = jnp.dot(h_mean, hcw_ref[...]) + hcb_ref[...]
    hbar_ref[...] = h_bar
    q0_ref[...] = h_bar + (jnp.dot(initw_ref[...], vww_ref[...]) + vwb_ref[...])


def _decoder_body(h_ref, rp_ref, hbar_ref, q0_ref, g_ref, pqw_ref, pqb_ref,
                  pv_ref, vww_ref, vwb_ref, lps_ref, chs_ref):
    bb = _BB_DEC
    h = h_ref[...]
    rp = rp_ref[...]
    hbar = hbar_ref[...]
    pqw = pqw_ref[...]
    pqb = pqb_ref[...]
    pv = pv_ref[...]
    vww = vww_ref[...]
    vwb = vwb_ref[...]
    iota_s = jax.lax.broadcasted_iota(jnp.int32, (bb, _S), 1)

    def body(i, carry):
        query, maskf, first, lps_acc, chs_acc = carry
        qp = jnp.dot(query, pqw) + pqb
        t = jnp.tanh(qp[:, None, :] + rp)
        u1 = jnp.dot(t.reshape(bb * _S, _D), pv)
        u = jnp.sum(u1.reshape(bb, _S, 1), axis=-1)
        logits = _CS * jnp.tanh(u)
        logits = jnp.where(maskf > 0.5, -100000.0, logits)
        z = logits + g_ref[i]
        zm = jnp.max(z, axis=-1, keepdims=True)
        cand = jnp.where(z == zm, iota_s, _S)
        chosen = jnp.min(cand, axis=-1, keepdims=True)
        lmax = jnp.max(logits, axis=-1, keepdims=True)
        sh = logits - lmax
        lsm = sh - jnp.log(_lane_sum(jnp.exp(sh)))
        onehot = iota_s == chosen
        oh_f = onehot.astype(jnp.float32)
        logprob = jnp.sum(jnp.where(onehot, lsm, 0.0), axis=-1, keepdims=True)
        maskf = jnp.maximum(maskf, oh_f)
        chosen_hs = jax.lax.dot_general(
            oh_f, h, (((1,), (1,)), ((0,), (0,))))
        first = jnp.where(i == 0, chosen_hs, first)
        cat = jnp.concatenate([first, chosen_hs], axis=-1)
        query = hbar + (jnp.dot(cat, vww) + vwb)
        lps_acc = jnp.where(iota_s == i, logprob, lps_acc)
        chs_acc = jnp.where(iota_s == i, chosen, chs_acc)
        return query, maskf, first, lps_acc, chs_acc

    init = (q0_ref[...],
            jnp.zeros((bb, _S), jnp.float32),
            jnp.zeros((bb, _D), jnp.float32),
            jnp.zeros((bb, _S), jnp.float32),
            jnp.zeros((bb, _S), jnp.int32))
    _, _, _, lps_acc, chs_acc = jax.lax.fori_loop(0, _S, body, init)
    lps_ref[...] = lps_acc
    chs_ref[...] = chs_acc


def _full(shape):
    return pl.BlockSpec(shape, lambda i: (0,) * len(shape))


def kernel(inputs, W_emb, init_w, in_proj_w, in_proj_b, out_proj_w, out_proj_b,
           ln1_g, ln1_b, ff_w1, ff_b1, ff_w2, ff_b2, ln2_g, ln2_b,
           hc_w, hc_b, vw_w, vw_b, pq_w, pq_b, pr_w, pr_b, p_v):
    inp2 = inputs.reshape(_B * _S, 2)
    base_key = jax.random.key(42)
    g_tab = jnp.stack([
        jax.random.gumbel(jax.random.fold_in(base_key, i), (_B, _S), jnp.float32)
        for i in range(_S)
    ])

    r1 = lambda a: a.reshape(1, -1)
    nb = _B // _BB_ENC
    enc_specs = [
        pl.BlockSpec((_BB_ENC * _S, 2), lambda i: (i, 0)),
        _full((_D, 3 * _D)), _full((1, 3 * _D)),
        _full((_D, _D)), _full((1, _D)),
        _full((1, _D)), _full((1, _D)),
        _full((_D, _FF)), _full((1, _FF)),
        _full((_FF, _D)), _full((1, _D)),
        _full((1, _D)), _full((1, _D)),
        _full((_D, _D)), _full((1, _D)),
        _full((1, 2 * _D)),
        _full((2 * _D, _D)), _full((1, _D)),
        _full((_D, _D)), _full((1, _D)),
        _full((2, _D)),
    ]
    enc_out_specs = [
        pl.BlockSpec((_BB_ENC, _S, _D), lambda i: (i, 0, 0)),
        pl.BlockSpec((_BB_ENC, _S, _D), lambda i: (i, 0, 0)),
        pl.BlockSpec((_BB_ENC, _D), lambda i: (i, 0)),
        pl.BlockSpec((_BB_ENC, _D), lambda i: (i, 0)),
    ]
    h, rp, hbar, q0 = pl.pallas_call(
        _encoder_body,
        grid=(nb,),
        in_specs=enc_specs,
        out_specs=enc_out_specs,
        out_shape=[
            jax.ShapeDtypeStruct((_B, _S, _D), jnp.float32),
            jax.ShapeDtypeStruct((_B, _S, _D), jnp.float32),
            jax.ShapeDtypeStruct((_B, _D), jnp.float32),
            jax.ShapeDtypeStruct((_B, _D), jnp.float32),
        ],
        compiler_params=pltpu.CompilerParams(
            dimension_semantics=("parallel",)),
    )(inp2, in_proj_w.T, r1(in_proj_b), out_proj_w.T, r1(out_proj_b),
      r1(ln1_g), r1(ln1_b), ff_w1.T, r1(ff_b1), ff_w2.T, r1(ff_b2),
      r1(ln2_g), r1(ln2_b), hc_w.T, r1(hc_b), r1(init_w), vw_w.T, r1(vw_b),
      pr_w.T, r1(pr_b), W_emb)

    nd = _B // _BB_DEC
    dec_specs = [
        pl.BlockSpec((_BB_DEC, _S, _D), lambda j: (j, 0, 0)),
        pl.BlockSpec((_BB_DEC, _S, _D), lambda j: (j, 0, 0)),
        pl.BlockSpec((_BB_DEC, _D), lambda j: (j, 0)),
        pl.BlockSpec((_BB_DEC, _D), lambda j: (j, 0)),
        pl.BlockSpec((_S, _BB_DEC, _S), lambda j: (0, j, 0)),
        _full((_D, _D)), _full((1, _D)),
        _full((_D, 1)),
        _full((2 * _D, _D)), _full((1, _D)),
    ]
    dec_out_specs = [
        pl.BlockSpec((_BB_DEC, _S), lambda j: (j, 0)),
        pl.BlockSpec((_BB_DEC, _S), lambda j: (j, 0)),
    ]
    lps, chs = pl.pallas_call(
        _decoder_body,
        grid=(nd,),
        in_specs=dec_specs,
        out_specs=dec_out_specs,
        out_shape=[
            jax.ShapeDtypeStruct((_B, _S), jnp.float32),
            jax.ShapeDtypeStruct((_B, _S), jnp.int32),
        ],
        compiler_params=pltpu.CompilerParams(
            dimension_semantics=("parallel",)),
    )(h, rp, hbar, q0, g_tab, pq_w.T, r1(pq_b), p_v.reshape(_D, 1), vw_w.T,
      r1(vw_b))
    return lps, chs
